# Initial kernel scaffold; baseline (speedup 1.0000x reference)
import jax
import jax.numpy as jnp
from jax.experimental import pallas as pl


def kernel(x, mapping_indices, lut):
    raise NotImplementedError("write your pallas kernel here")


# speedup vs baseline: 4.8100x; 4.8100x over previous
import functools

import jax
import jax.numpy as jnp
from jax import lax
from jax.experimental import pallas as pl
from jax.experimental.pallas import tpu as pltpu
from jax.experimental.pallas import tpu_sc as plsc

_NC = 2
_NS = 16
_L = 16


@functools.cache
def _build_sc_kernel(B, IN, O, QN):
    NW = _NC * _NS
    R = B // NW
    NQ = O // QN
    NCH = QN // _L

    mesh = plsc.VectorSubcoreMesh(core_axis_name="c", subcore_axis_name="s")

    @functools.partial(
        pl.kernel,
        out_type=jax.ShapeDtypeStruct((B, O), jnp.float32),
        mesh=mesh,
        compiler_params=pltpu.CompilerParams(needs_layout_passes=False),
        scratch_types=[
            pltpu.VMEM((R, IN), jnp.float32),
            pltpu.VMEM((4, QN), jnp.int32),
            pltpu.VMEM((16, QN), jnp.float32),
            pltpu.VMEM((R, QN), jnp.float32),
        ],
    )
    def sc_kernel(x_hbm, idxt_hbm, lutt_hbm, out_hbm, x_v, idx_v, lut_v, out_v):
        wid = lax.axis_index("s") * _NC + lax.axis_index("c")
        b0 = wid * R
        pltpu.sync_copy(x_hbm.at[pl.ds(b0, R)], x_v)

        for q in range(NQ):
            pltpu.sync_copy(idxt_hbm.at[:, pl.ds(q * QN, QN)], idx_v)
            pltpu.sync_copy(lutt_hbm.at[:, pl.ds(q * QN, QN)], lut_v)

            def chunk_body(c, _):
                o0 = c * _L
                i0 = idx_v[0, pl.ds(o0, _L)]
                i1 = idx_v[1, pl.ds(o0, _L)]
                i2 = idx_v[2, pl.ds(o0, _L)]
                i3 = idx_v[3, pl.ds(o0, _L)]
                lv = [lut_v[k, pl.ds(o0, _L)] for k in range(16)]

                def batch_body(b, _):
                    bvec = jnp.full((_L,), b, dtype=jnp.int32)
                    s0 = plsc.load_gather(x_v, [bvec, i0])
                    s1 = plsc.load_gather(x_v, [bvec, i1])
                    s2 = plsc.load_gather(x_v, [bvec, i2])
                    s3 = plsc.load_gather(x_v, [bvec, i3])
                    t3 = s0 * s1
                    t1 = s0 - t3
                    t2 = s1 - t3
                    t0 = (1.0 - s0) - t2
                    u3 = s2 * s3
                    u1 = s2 - u3
                    u2 = s3 - u3
                    u0 = (1.0 - s2) - u2
                    v0 = lv[0] * t0 + lv[1] * t1 + lv[2] * t2 + lv[3] * t3
                    v1 = lv[4] * t0 + lv[5] * t1 + lv[6] * t2 + lv[7] * t3
                    v2 = lv[8] * t0 + lv[9] * t1 + lv[10] * t2 + lv[11] * t3
                    v3 = lv[12] * t0 + lv[13] * t1 + lv[14] * t2 + lv[15] * t3
                    out_v[b, pl.ds(o0, _L)] = u0 * v0 + u1 * v1 + u2 * v2 + u3 * v3
                    return 0

                lax.fori_loop(0, R, batch_body, 0)
                return 0

            lax.fori_loop(0, NCH, chunk_body, 0)
            pltpu.sync_copy(out_v, out_hbm.at[pl.ds(b0, R), pl.ds(q * QN, QN)])

    return sc_kernel


def kernel(x, mapping_indices, lut):
    B, IN = x.shape
    O, n = mapping_indices.shape
    assert n == 4 and lut.shape == (O, 16)
    idxt = mapping_indices.T.astype(jnp.int32)
    lutt = lut.astype(jnp.float32).T
    return _build_sc_kernel(B, IN, O, 512)(x.astype(jnp.float32), idxt, lutt)
